# baseline (speedup 1.0000x reference)
import jax
import jax.numpy as jnp
from jax.experimental import pallas as pl
from jax.experimental.pallas import tpu as pltpu

VOCAB = 100000
EMBED = 64
BATCH = 1024
KB = 1024
NB = 4096


def _fc1_body(xt_ref, w1_ref, b1_ref, ht_ref, acc_ref):
    k = pl.program_id(0)

    @pl.when(k == 0)
    def _init():
        acc_ref[...] = jnp.zeros_like(acc_ref)

    xv = xt_ref[...]
    wv = w1_ref[...]

    @pl.when(k < pl.num_programs(0) - 1)
    def _full():
        acc_ref[...] += jax.lax.dot_general(
            wv, xv,
            dimension_numbers=(((1,), (0,)), ((), ())),
            preferred_element_type=jnp.float32,
        )

    @pl.when(k == pl.num_programs(0) - 1)
    def _partial():
        valid = VOCAB - k * KB
        row = jax.lax.broadcasted_iota(jnp.int32, (KB, 1), 0)
        col = jax.lax.broadcasted_iota(jnp.int32, (1, KB), 1)
        xm = jnp.where(row < valid, xv, 0.0)
        wm = jnp.where(col < valid, wv, 0.0)
        acc_ref[...] += jax.lax.dot_general(
            wm, xm,
            dimension_numbers=(((1,), (0,)), ((), ())),
            preferred_element_type=jnp.float32,
        )

    @pl.when(k == pl.num_programs(0) - 1)
    def _fin():
        ones_row = jnp.ones((1, BATCH), dtype=jnp.float32)
        b1_row = b1_ref[...].reshape(1, EMBED)
        ht_ref[...] = acc_ref[...] + jax.lax.dot_general(
            b1_row, ones_row,
            dimension_numbers=(((0,), (0,)), ((), ())),
            preferred_element_type=jnp.float32,
        )


def _fc2_body(ht_ref, w2t_ref, b2_ref, ot_ref):
    ones_row = jnp.ones((1, BATCH), dtype=jnp.float32)
    b2_row = b2_ref[...].reshape(1, NB)
    ot_ref[...] = jax.lax.dot_general(
        w2t_ref[...], ht_ref[...],
        dimension_numbers=(((0,), (0,)), ((), ())),
        preferred_element_type=jnp.float32,
    ) + jax.lax.dot_general(
        b2_row, ones_row,
        dimension_numbers=(((0,), (0,)), ((), ())),
        preferred_element_type=jnp.float32,
    )


def kernel(x, W1, b1, W2, b2):
    xt = x.T
    w2t = W2.T

    ht = pl.pallas_call(
        _fc1_body,
        grid=(pl.cdiv(VOCAB, KB),),
        in_specs=[
            pl.BlockSpec((KB, BATCH), lambda k: (k, 0)),
            pl.BlockSpec((EMBED, KB), lambda k: (0, k)),
            pl.BlockSpec((EMBED,), lambda k: (0,)),
        ],
        out_specs=pl.BlockSpec((EMBED, BATCH), lambda k: (0, 0)),
        out_shape=jax.ShapeDtypeStruct((EMBED, BATCH), jnp.float32),
        scratch_shapes=[pltpu.VMEM((EMBED, BATCH), jnp.float32)],
        compiler_params=pltpu.CompilerParams(
            dimension_semantics=("arbitrary",),
        ),
    )(xt, W1, b1)

    ot = pl.pallas_call(
        _fc2_body,
        grid=(pl.cdiv(VOCAB, NB),),
        in_specs=[
            pl.BlockSpec((EMBED, BATCH), lambda n: (0, 0)),
            pl.BlockSpec((EMBED, NB), lambda n: (0, n)),
            pl.BlockSpec((NB,), lambda n: (n,)),
        ],
        out_specs=pl.BlockSpec((NB, BATCH), lambda n: (n, 0)),
        out_shape=jax.ShapeDtypeStruct((VOCAB, BATCH), jnp.float32),
        compiler_params=pltpu.CompilerParams(
            dimension_semantics=("parallel",),
            vmem_limit_bytes=65011712,
        ),
    )(ht, w2t, b2)

    return ot.T


# speedup vs baseline: 1.0978x; 1.0978x over previous
import jax
import jax.numpy as jnp
from jax.experimental import pallas as pl
from jax.experimental.pallas import tpu as pltpu

VOCAB = 100000
EMBED = 64
BATCH = 1024
KB = 2048
NB = 4096


def _fc1_body(xt_ref, w1_ref, b1_ref, ht_ref, acc_ref):
    k = pl.program_id(0)

    @pl.when(k == 0)
    def _init():
        acc_ref[...] = jnp.zeros_like(acc_ref)

    xv = xt_ref[...]
    wv = w1_ref[...]

    @pl.when(k < pl.num_programs(0) - 1)
    def _full():
        acc_ref[...] += jax.lax.dot_general(
            wv, xv,
            dimension_numbers=(((1,), (0,)), ((), ())),
            preferred_element_type=jnp.float32,
        )

    @pl.when(k == pl.num_programs(0) - 1)
    def _partial():
        valid = VOCAB - k * KB
        row = jax.lax.broadcasted_iota(jnp.int32, (KB, 1), 0)
        col = jax.lax.broadcasted_iota(jnp.int32, (1, KB), 1)
        xm = jnp.where(row < valid, xv, 0.0)
        wm = jnp.where(col < valid, wv, 0.0)
        acc_ref[...] += jax.lax.dot_general(
            wm, xm,
            dimension_numbers=(((1,), (0,)), ((), ())),
            preferred_element_type=jnp.float32,
        )

    @pl.when(k == pl.num_programs(0) - 1)
    def _fin():
        ones_row = jnp.ones((1, BATCH), dtype=jnp.float32)
        b1_row = b1_ref[...].reshape(1, EMBED)
        ht_ref[...] = acc_ref[...] + jax.lax.dot_general(
            b1_row, ones_row,
            dimension_numbers=(((0,), (0,)), ((), ())),
            preferred_element_type=jnp.float32,
        )


def _fc2_body(ht_ref, w2t_ref, b2_ref, ot_ref):
    ones_row = jnp.ones((1, BATCH), dtype=jnp.float32)
    b2_row = b2_ref[...].reshape(1, NB)
    ot_ref[...] = jax.lax.dot_general(
        w2t_ref[...], ht_ref[...],
        dimension_numbers=(((0,), (0,)), ((), ())),
        preferred_element_type=jnp.float32,
    ) + jax.lax.dot_general(
        b2_row, ones_row,
        dimension_numbers=(((0,), (0,)), ((), ())),
        preferred_element_type=jnp.float32,
    )


def kernel(x, W1, b1, W2, b2):
    xt = x.T
    w2t = W2.T

    ht = pl.pallas_call(
        _fc1_body,
        grid=(pl.cdiv(VOCAB, KB),),
        in_specs=[
            pl.BlockSpec((KB, BATCH), lambda k: (k, 0)),
            pl.BlockSpec((EMBED, KB), lambda k: (0, k)),
            pl.BlockSpec((EMBED,), lambda k: (0,)),
        ],
        out_specs=pl.BlockSpec((EMBED, BATCH), lambda k: (0, 0)),
        out_shape=jax.ShapeDtypeStruct((EMBED, BATCH), jnp.float32),
        scratch_shapes=[pltpu.VMEM((EMBED, BATCH), jnp.float32)],
        compiler_params=pltpu.CompilerParams(
            dimension_semantics=("arbitrary",),
        ),
    )(xt, W1, b1)

    ot = pl.pallas_call(
        _fc2_body,
        grid=(pl.cdiv(VOCAB, NB),),
        in_specs=[
            pl.BlockSpec((EMBED, BATCH), lambda n: (0, 0)),
            pl.BlockSpec((EMBED, NB), lambda n: (0, n)),
            pl.BlockSpec((NB,), lambda n: (n,)),
        ],
        out_specs=pl.BlockSpec((NB, BATCH), lambda n: (n, 0)),
        out_shape=jax.ShapeDtypeStruct((VOCAB, BATCH), jnp.float32),
        compiler_params=pltpu.CompilerParams(
            dimension_semantics=("parallel",),
        ),
    )(ht, w2t, b2)

    return ot.T
